# baseline (speedup 1.0000x reference)
import functools

import jax
import jax.numpy as jnp
from jax import lax
from jax.experimental import pallas as pl
from jax.experimental.pallas import tpu as pltpu
from jax.experimental.pallas import tpu_sc as plsc

_NC = 2
_NS = 16
_NW = _NC * _NS

_LANES = 16
_CHUNK = 128


def _tw_body(t_ref, wt_ref, o_ref):
    o_ref[:] = jnp.sum(t_ref[:] * wt_ref[:], axis=0)


def _compute_tw(table, W):
    V, D = table.shape
    C = 131072
    grid = (V + C - 1) // C
    return pl.pallas_call(
        _tw_body,
        grid=(grid,),
        in_specs=[
            pl.BlockSpec((D, C), lambda i: (0, i)),
            pl.BlockSpec((D, 1), lambda i: (0, 0)),
        ],
        out_specs=pl.BlockSpec((C,), lambda i: (i,)),
        out_shape=jax.ShapeDtypeStruct((V,), jnp.float32),
    )(table.T, W.T)


def _make_relay_kernel(B, L):
    n_total_chunks = B // _CHUNK
    per_w = n_total_chunks // _NW

    mesh = plsc.VectorSubcoreMesh(core_axis_name="c", subcore_axis_name="s")

    @functools.partial(
        pl.kernel,
        mesh=mesh,
        out_type=jax.ShapeDtypeStruct((n_total_chunks, L, _CHUNK), jnp.int32),
        scratch_types=[
            pltpu.VMEM((L, _CHUNK), jnp.int32),
        ],
        compiler_params=pltpu.CompilerParams(needs_layout_passes=False),
    )
    def relay(xt_hbm, xp_hbm, stage_v):
        wid = lax.axis_index("s") * _NC + lax.axis_index("c")

        def do_chunk(c, _):
            g = wid * per_w + c
            pltpu.sync_copy(xt_hbm.at[:, pl.ds(g * _CHUNK, _CHUNK)], stage_v)
            pltpu.sync_copy(stage_v, xp_hbm.at[g])
            return 0

        lax.fori_loop(0, per_w, do_chunk, 0, unroll=True)

    return relay


def _make_sc_kernel(B, L, V):
    rows_per_w = B // _NW
    n_chunks = rows_per_w // _CHUNK
    n_groups = _CHUNK // _LANES
    l_half = L // 2
    flat = l_half * _CHUNK
    n_rounds = n_chunks * 2

    mesh = plsc.VectorSubcoreMesh(core_axis_name="c", subcore_axis_name="s")

    @functools.partial(
        pl.kernel,
        mesh=mesh,
        out_type=jax.ShapeDtypeStruct((B,), jnp.float32),
        scratch_types=[
            pltpu.VMEM((flat,), jnp.int32),
            pltpu.VMEM((flat,), jnp.int32),
            pltpu.VMEM((flat,), jnp.float32),
            pltpu.VMEM((flat,), jnp.float32),
            pltpu.VMEM((rows_per_w,), jnp.float32),
            pltpu.VMEM((_LANES,), jnp.float32),
            pltpu.VMEM_SHARED((V,), jnp.float32),
            pltpu.SemaphoreType.DMA,
            pltpu.SemaphoreType.DMA,
        ],
    )
    def sc_kernel(xp_hbm, tw_hbm, b_hbm, out_hbm,
                  idx_a, idx_b, vals_a, vals_b, out_v, b_v, tw_s,
                  sem_a, sem_b):
        wid = lax.axis_index("s") * _NC + lax.axis_index("c")
        pltpu.sync_copy(b_hbm, b_v)
        bias = b_v[:]
        inv_l = jnp.float32(1.0 / L)

        @pl.when(lax.axis_index("s") == 0)
        def _stage():
            pltpu.sync_copy(tw_hbm, tw_s)

        plsc.subcore_barrier()

        idx = (idx_a, idx_b)
        vals = (vals_a, vals_b)
        sems = (sem_a, sem_b)
        base = wid * n_chunks * L * _CHUNK

        def fetch(r):
            p = r % 2
            pltpu.sync_copy(xp_hbm.at[pl.ds(base + r * flat, flat)], idx[p])
            return pltpu.async_copy(tw_s.at[idx[p]], vals[p], sems[p])

        def accum(r, acc):
            p = r % 2
            v = vals[p]

            def body(l, a):
                off = l * _CHUNK
                return tuple(
                    a[j] + v[pl.ds(off + j * _LANES, _LANES)]
                    for j in range(n_groups)
                )

            return lax.fori_loop(0, l_half, body, acc)

        zero = jnp.zeros((_LANES,), jnp.float32)
        copies = [None, None]
        copies[0] = fetch(0)
        for c in range(n_chunks):
            acc = (zero,) * n_groups
            for h in range(2):
                r = c * 2 + h
                if r + 1 < n_rounds:
                    copies[(r + 1) % 2] = fetch(r + 1)
                copies[r % 2].wait()
                acc = accum(r, acc)
            for j in range(n_groups):
                z = acc[j] * inv_l + bias
                s = 1.0 / (1.0 + jnp.exp(-z))
                out_v[pl.ds(c * _CHUNK + j * _LANES, _LANES)] = s

        pltpu.sync_copy(out_v, out_hbm.at[pl.ds(wid * rows_per_w, rows_per_w)])

    return sc_kernel


@jax.jit
def kernel(x, table, W, b):
    B, L = x.shape
    V, D = table.shape

    tw = _compute_tw(table, W)
    b_vec = jnp.broadcast_to(b.astype(jnp.float32), (_LANES,))

    xt = x.astype(jnp.int32).T
    xp = _make_relay_kernel(B, L)(xt)
    xp_flat = xp.reshape(B * L)

    out = _make_sc_kernel(B, L, V)(xp_flat, tw, b_vec)
    return out.reshape(B, 1)


# speedup vs baseline: 1.0090x; 1.0090x over previous
import functools

import jax
import jax.numpy as jnp
from jax import lax
from jax.experimental import pallas as pl
from jax.experimental.pallas import tpu as pltpu
from jax.experimental.pallas import tpu_sc as plsc

_NC = 2
_NS = 16
_NW = _NC * _NS

_LANES = 16
_CHUNK = 128


def _tw_body(t_ref, wt_ref, o_ref):
    o_ref[:] = jnp.sum(t_ref[:] * wt_ref[:], axis=0)


def _compute_tw(table, W):
    V, D = table.shape
    C = 131072
    grid = (V + C - 1) // C
    return pl.pallas_call(
        _tw_body,
        grid=(grid,),
        in_specs=[
            pl.BlockSpec((D, C), lambda i: (0, i)),
            pl.BlockSpec((D, 1), lambda i: (0, 0)),
        ],
        out_specs=pl.BlockSpec((C,), lambda i: (i,)),
        out_shape=jax.ShapeDtypeStruct((V,), jnp.float32),
    )(table.T, W.T)


def _make_relay_kernel(B, L):
    n_total_chunks = B // _CHUNK
    per_w = n_total_chunks // _NW

    mesh = plsc.VectorSubcoreMesh(core_axis_name="c", subcore_axis_name="s")

    @functools.partial(
        pl.kernel,
        mesh=mesh,
        out_type=jax.ShapeDtypeStruct((n_total_chunks, L, _CHUNK), jnp.int32),
        scratch_types=[
            pltpu.VMEM((L, _CHUNK), jnp.int32),
        ],
        compiler_params=pltpu.CompilerParams(needs_layout_passes=False),
    )
    def relay(xt_hbm, xp_hbm, stage_v):
        wid = lax.axis_index("s") * _NC + lax.axis_index("c")

        def do_chunk(c, _):
            g = wid * per_w + c
            pltpu.sync_copy(xt_hbm.at[:, pl.ds(g * _CHUNK, _CHUNK)], stage_v)
            pltpu.sync_copy(stage_v, xp_hbm.at[g])
            return 0

        lax.fori_loop(0, per_w, do_chunk, 0, unroll=True)

    return relay


def _make_sc_kernel(B, L, V):
    rows_per_w = B // _NW
    n_chunks = rows_per_w // _CHUNK
    n_groups = _CHUNK // _LANES
    n_buf = 4
    l_q = L // n_buf
    flat = l_q * _CHUNK
    n_rounds = n_chunks * n_buf

    mesh = plsc.VectorSubcoreMesh(core_axis_name="c", subcore_axis_name="s")

    @functools.partial(
        pl.kernel,
        mesh=mesh,
        out_type=jax.ShapeDtypeStruct((B,), jnp.float32),
        scratch_types=[
            [pltpu.VMEM((flat,), jnp.int32)] * n_buf,
            [pltpu.VMEM((flat,), jnp.float32)] * n_buf,
            pltpu.VMEM((rows_per_w,), jnp.float32),
            pltpu.VMEM((_LANES,), jnp.float32),
            pltpu.VMEM_SHARED((V,), jnp.float32),
            [pltpu.SemaphoreType.DMA] * n_buf,
        ],
    )
    def sc_kernel(xp_hbm, tw_hbm, b_hbm, out_hbm,
                  idx, vals, out_v, b_v, tw_s, sems):
        wid = lax.axis_index("s") * _NC + lax.axis_index("c")
        pltpu.sync_copy(b_hbm, b_v)
        bias = b_v[:]
        inv_l = jnp.float32(1.0 / L)

        @pl.when(lax.axis_index("s") == 0)
        def _stage():
            pltpu.sync_copy(tw_hbm, tw_s)

        plsc.subcore_barrier()

        base = wid * n_chunks * L * _CHUNK

        def fetch(r):
            p = r % n_buf
            pltpu.sync_copy(xp_hbm.at[pl.ds(base + r * flat, flat)], idx[p])
            return pltpu.async_copy(tw_s.at[idx[p]], vals[p], sems[p])

        def accum(r, acc):
            v = vals[r % n_buf]

            def body(l, a):
                off = l * _CHUNK
                return tuple(
                    a[j] + v[pl.ds(off + j * _LANES, _LANES)]
                    for j in range(n_groups)
                )

            return lax.fori_loop(0, l_q, body, acc)

        zero = jnp.zeros((_LANES,), jnp.float32)
        copies = [fetch(r) for r in range(n_buf - 1)]
        copies.append(None)
        for c in range(n_chunks):
            acc = (zero,) * n_groups
            for h in range(n_buf):
                r = c * n_buf + h
                if r + n_buf - 1 < n_rounds:
                    copies[(r + n_buf - 1) % n_buf] = fetch(r + n_buf - 1)
                copies[r % n_buf].wait()
                acc = accum(r, acc)
            for j in range(n_groups):
                z = acc[j] * inv_l + bias
                s = 1.0 / (1.0 + jnp.exp(-z))
                out_v[pl.ds(c * _CHUNK + j * _LANES, _LANES)] = s

        pltpu.sync_copy(out_v, out_hbm.at[pl.ds(wid * rows_per_w, rows_per_w)])

    return sc_kernel


@jax.jit
def kernel(x, table, W, b):
    B, L = x.shape
    V, D = table.shape

    tw = _compute_tw(table, W)
    b_vec = jnp.broadcast_to(b.astype(jnp.float32), (_LANES,))

    xt = x.astype(jnp.int32).T
    xp = _make_relay_kernel(B, L)(xt)
    xp_flat = xp.reshape(B * L)

    out = _make_sc_kernel(B, L, V)(xp_flat, tw, b_vec)
    return out.reshape(B, 1)
